# baseline (speedup 1.0000x reference)
import functools
import math

import jax
import jax.numpy as jnp
from jax import lax
from jax.experimental import pallas as pl
from jax.experimental.pallas import tpu as pltpu
from jax.experimental.pallas import tpu_sc as plsc

D = 64
LANES = 16
NC, NS = 2, 16
NW = NC * NS

V = 1000000
VBLK = 128
NFULL = V // VBLK
TS_ROWS = 500032
_MAIN = (NFULL // NW) * NW
NB1 = 4

B, S = 4096, 200
BW = B // NW
NB2 = 4


def _iota16():
    return lax.iota(jnp.int32, 16)


def _transpose_table(tt, tail):
    mesh = plsc.VectorSubcoreMesh(core_axis_name="c", subcore_axis_name="s")

    @functools.partial(
        pl.kernel,
        out_type=jax.ShapeDtypeStruct((TS_ROWS, VBLK), jnp.float32),
        mesh=mesh,
        scratch_types=[
            pltpu.VMEM((NB1, D, VBLK), jnp.float32),
            pltpu.VMEM((NB1, D, VBLK), jnp.float32),
            pltpu.SemaphoreType.DMA,
            pltpu.SemaphoreType.DMA,
        ],
        compiler_params=pltpu.CompilerParams(needs_layout_passes=False),
    )
    def k(tt_hbm, tail_hbm, ts_hbm, ibuf, obuf, rsem, wsem):
        c = lax.axis_index("c")
        s = lax.axis_index("s")
        wid = s * NC + c

        def rd(cb, b):
            pltpu.async_copy(tt_hbm.at[:, pl.ds(cb * VBLK, VBLK)], ibuf.at[b], rsem)

        def rd_wait(b):
            pltpu.make_async_copy(
                tt_hbm.at[:, pl.ds(0, VBLK)], ibuf.at[b], rsem
            ).wait()

        def wr(cb, b):
            pltpu.async_copy(obuf.at[b], ts_hbm.at[pl.ds(cb * D, D)], wsem)

        def wr_wait(b):
            pltpu.make_async_copy(
                obuf.at[b], ts_hbm.at[pl.ds(0, D)], wsem
            ).wait()

        def transpose_block(b):
            def tb(vq, carry):
                for u in range(8):
                    vv = vq * 8 + u
                    col = jnp.broadcast_to(vv, (16,))
                    base = (vv & 1) * D
                    q = vv >> 1
                    for kk in range(D // LANES):
                        rows = kk * LANES + _iota16()
                        vals = plsc.load_gather(ibuf.at[b], [rows, col])
                        obuf[b, q, pl.ds(base + kk * LANES, LANES)] = vals
                return carry

            lax.fori_loop(0, VBLK // 8, tb, 0)

        for b in range(NB1):
            rd(NW * b + wid, b)

        ngroups = _MAIN // NW // NB1

        def group(g, carry):
            for b in range(NB1):
                t = g * NB1 + b
                cb = NW * t + wid
                rd_wait(b)

                @pl.when(g > 0)
                def _(b=b):
                    wr_wait(b)

                transpose_block(b)
                wr(cb, b)

                @pl.when(g + 1 < ngroups)
                def _(b=b, t=t):
                    rd(NW * (t + NB1) + wid, b)

            return carry

        lax.fori_loop(0, ngroups, group, 0)
        for b in range(NB1):
            wr_wait(b)

        @pl.when(wid < NFULL - _MAIN)
        def _():
            cb = _MAIN + wid
            pltpu.sync_copy(tt_hbm.at[:, pl.ds(cb * VBLK, VBLK)], ibuf.at[0])
            transpose_block(0)
            pltpu.sync_copy(obuf.at[0], ts_hbm.at[pl.ds(cb * D, D)])

        @pl.when(wid == NW - 1)
        def _():
            pltpu.sync_copy(tail_hbm, ibuf.at[0, pl.ds(0, D // 2)])
            pltpu.sync_copy(
                ibuf.at[0, pl.ds(0, D // 2)],
                ts_hbm.at[pl.ds(NFULL * D, D // 2)],
            )

    return k(tt, tail)


def _gather_scaled(xt, ts):
    scale = jnp.float32(math.sqrt(float(D)))
    mesh = plsc.VectorSubcoreMesh(core_axis_name="c", subcore_axis_name="s")

    @functools.partial(
        pl.kernel,
        out_type=jax.ShapeDtypeStruct((S, D, B), jnp.float32),
        mesh=mesh,
        scratch_types=[
            pltpu.VMEM((S, BW), jnp.int32),
            pltpu.VMEM((NB2, BW), jnp.int32),
            pltpu.VMEM((NB2, BW, VBLK), jnp.float32),
            pltpu.VMEM((NB2, D, BW), jnp.float32),
            pltpu.SemaphoreType.DMA,
            pltpu.SemaphoreType.DMA,
        ],
        compiler_params=pltpu.CompilerParams(needs_layout_passes=False),
    )
    def k(xt_hbm, ts_hbm, oq_hbm, idx_v, qbuf, gbuf, obuf, gsem, osem):
        c = lax.axis_index("c")
        s = lax.axis_index("s")
        wid = s * NC + c
        col0 = wid * BW
        pltpu.sync_copy(xt_hbm.at[:, pl.ds(col0, BW)], idx_v)

        def fire_gather(si, b):
            for kk in range(BW // LANES):
                sl = pl.ds(kk * LANES, LANES)
                qbuf[b, sl] = lax.shift_right_logical(idx_v[si, sl], 1)
            pltpu.async_copy(ts_hbm.at[qbuf.at[b]], gbuf.at[b], gsem)

        def gather_wait(b):
            pltpu.make_async_copy(
                ts_hbm.at[pl.ds(0, BW)], gbuf.at[b], gsem
            ).wait()

        def owrite(si, b):
            pltpu.async_copy(
                obuf.at[b], oq_hbm.at[si, :, pl.ds(col0, BW)], osem
            )

        def owrite_wait(b):
            pltpu.make_async_copy(
                obuf.at[b], oq_hbm.at[0, :, pl.ds(col0, BW)], osem
            ).wait()

        def compute(si, b):
            rvecs, cbases = [], []
            for k8 in range(BW // LANES):
                sl = pl.ds(k8 * LANES, LANES)
                rvecs.append(k8 * LANES + _iota16())
                cbases.append((idx_v[si, sl] & 1) * D)

            def dstep(dq, carry):
                for j in range(4):
                    d = dq * 4 + j
                    for k8 in range(BW // LANES):
                        vals = plsc.load_gather(
                            gbuf.at[b], [rvecs[k8], cbases[k8] + d]
                        )
                        obuf[b, d, pl.ds(k8 * LANES, LANES)] = vals * scale
                return carry

            lax.fori_loop(0, D // 4, dstep, 0)

        for b in range(NB2):
            fire_gather(b, b)

        ngroups = S // NB2

        def group(g, carry):
            for b in range(NB2):
                si = g * NB2 + b
                gather_wait(b)

                @pl.when(g > 0)
                def _(b=b):
                    owrite_wait(b)

                compute(si, b)
                owrite(si, b)

                @pl.when(g + 1 < ngroups)
                def _(si=si, b=b):
                    fire_gather(si + NB2, b)

            return carry

        lax.fori_loop(0, ngroups, group, 0)
        for b in range(NB2):
            owrite_wait(b)

    return k(xt, ts)


def kernel(x, table):
    xt = jnp.transpose(x).astype(jnp.int32)
    tt = jnp.transpose(table)
    tail = jnp.reshape(table[NFULL * VBLK:], (D // 2, VBLK))
    ts = _transpose_table(tt, tail)
    oq = _gather_scaled(xt, ts)
    return jnp.transpose(oq, (2, 0, 1))


# speedup vs baseline: 2.3255x; 2.3255x over previous
import functools
import math

import jax
import jax.numpy as jnp
from jax import lax
from jax.experimental import pallas as pl
from jax.experimental.pallas import tpu as pltpu
from jax.experimental.pallas import tpu_sc as plsc

D = 64
LANES = 16
CHUNK = 128
NBUF = 4
NC, NS = 2, 16
NW = NC * NS


def _emb_sc(x_rows, table):
    n_rows = x_rows.shape[0]
    rows_per_w = n_rows // NW
    ngroups = rows_per_w // NBUF
    scale = jnp.float32(math.sqrt(float(D)))
    mesh = plsc.VectorSubcoreMesh(core_axis_name="c", subcore_axis_name="s")

    @functools.partial(
        pl.kernel,
        out_type=jax.ShapeDtypeStruct((n_rows * CHUNK, D), jnp.float32),
        mesh=mesh,
        scratch_types=[
            pltpu.VMEM((rows_per_w, CHUNK), jnp.int32),
            pltpu.VMEM((NBUF, CHUNK, D), jnp.float32),
            pltpu.VMEM((NBUF, CHUNK, D), jnp.float32),
            pltpu.SemaphoreType.DMA,
            pltpu.SemaphoreType.DMA,
        ],
        compiler_params=pltpu.CompilerParams(use_tc_tiling_on_sc=False),
    )
    def k(x_hbm, table_hbm, out_hbm, idx_v, inb, outb, gsem, osem):
        c = lax.axis_index("c")
        s = lax.axis_index("s")
        wid = s * NC + c
        row0 = wid * rows_per_w
        pltpu.sync_copy(x_hbm.at[pl.ds(row0, rows_per_w)], idx_v)

        def gather(j, b):
            pltpu.async_copy(table_hbm.at[idx_v.at[j]], inb.at[b], gsem)

        def gather_wait(j, b):
            del j
            pltpu.make_async_copy(
                table_hbm.at[pl.ds(0, CHUNK)], inb.at[b], gsem
            ).wait()

        def out_copy(j, b):
            pltpu.async_copy(
                outb.at[b], out_hbm.at[pl.ds((row0 + j) * CHUNK, CHUNK)], osem
            )

        def out_wait(j, b):
            pltpu.make_async_copy(
                outb.at[b], out_hbm.at[pl.ds((row0 + j) * CHUNK, CHUNK)], osem
            ).wait()

        for b in range(NBUF):
            gather(b, b)

        def group(g, carry):
            j0 = g * NBUF
            for b in range(NBUF):
                j = j0 + b
                gather_wait(j, b)

                @pl.when(g > 0)
                def _(b=b, j=j):
                    out_wait(j - NBUF, b)

                @functools.partial(plsc.parallel_loop, 0, CHUNK, unroll=4)
                def _(r, b=b):
                    for kk in range(D // LANES):
                        sl = pl.ds(kk * LANES, LANES)
                        outb[b, r, sl] = inb[b, r, sl] * scale

                out_copy(j, b)

                @pl.when(g + 1 < ngroups)
                def _(b=b, j=j):
                    gather(j + NBUF, b)

            return carry

        lax.fori_loop(0, ngroups, group, 0)

        for b in range(NBUF):
            out_wait((ngroups - 1) * NBUF + b, b)

    return k(x_rows, table)


def kernel(x, table):
    b, s = x.shape
    n = b * s
    x_rows = x.reshape(n // CHUNK, CHUNK).astype(jnp.int32)
    out = _emb_sc(x_rows, table)
    return out.reshape(b, s, D)
